# baseline (speedup 1.0000x reference)
import functools

import jax
import jax.numpy as jnp
from jax import lax
from jax.experimental import pallas as pl
from jax.experimental.pallas import tpu as pltpu
from jax.experimental.pallas import tpu_sc as plsc

VOCAB = 1000000
D = 64
S = 50
NB = 16384 // 128
NBLK = S * NB
NW = 32
BLK_PER_W = NBLK // NW
SCALE = float(D) ** 0.5

_MESH = plsc.VectorSubcoreMesh(core_axis_name="c", subcore_axis_name="s")


@functools.partial(
    pl.kernel,
    out_type=jax.ShapeDtypeStruct((S, 8, NB, 1024), jnp.float32),
    mesh=_MESH,
    compiler_params=pltpu.CompilerParams(
        use_tc_tiling_on_sc=False, needs_layout_passes=False),
    scratch_types=[
        pltpu.VMEM((BLK_PER_W, 128), jnp.int32),
        pltpu.VMEM((128, D), jnp.float32),
        pltpu.VMEM((128, D), jnp.float32),
        pltpu.VMEM((D * 128,), jnp.float32),
        pltpu.SemaphoreType.DMA,
        pltpu.SemaphoreType.DMA,
        pltpu.SemaphoreType.DMA,
    ],
)
def _emb_lookup(xb_hbm, table_hbm, out_hbm, idx_v, g0, g1, ob, semg0, semg1,
                sems):
    wid = lax.axis_index("s") * 2 + lax.axis_index("c")
    base_blk = wid * BLK_PER_W

    pltpu.sync_copy(xb_hbm.at[pl.ds(base_blk, BLK_PER_W)], idx_v)

    colvs = [(lax.iota(jnp.int32, 16) + 16 * k) * 128 for k in range(4)]

    def process(gbuf, n):
        blk = base_blk + n
        s = blk // NB
        j = blk % NB

        def row(p, carry):
            for k in range(4):
                v = gbuf[p, pl.ds(16 * k, 16)] * SCALE
                plsc.store_scatter(ob, [colvs[k] + p], v)
            return carry

        lax.fori_loop(0, 128, row, 0, unroll=2)

        for i in range(8):
            pltpu.async_copy(ob.at[pl.ds(1024 * i, 1024)],
                             out_hbm.at[s, i, j], sems)
        for i in range(8):
            pltpu.make_async_copy(ob.at[pl.ds(1024 * i, 1024)],
                                  out_hbm.at[s, i, j], sems).wait()

    pltpu.async_copy(table_hbm.at[idx_v.at[0]], g0, semg0)

    def pair(g, carry):
        n0 = 2 * g
        pltpu.async_copy(table_hbm.at[idx_v.at[n0 + 1]], g1, semg1)
        pltpu.make_async_copy(table_hbm.at[idx_v.at[0]], g0, semg0).wait()
        process(g0, n0)
        nxt = jnp.minimum(n0 + 2, BLK_PER_W - 1)
        pltpu.async_copy(table_hbm.at[idx_v.at[nxt]], g0, semg0)
        pltpu.make_async_copy(table_hbm.at[idx_v.at[0]], g1, semg1).wait()
        process(g1, n0 + 1)
        return carry

    lax.fori_loop(0, BLK_PER_W // 2, pair, 0)

    pltpu.make_async_copy(table_hbm.at[idx_v.at[0]], g0, semg0).wait()


def kernel(x, table):
    xb = x.T.reshape(S, NB, 128).reshape(NBLK, 128).astype(jnp.int32)
    t2 = jax.lax.optimization_barrier(table.reshape(VOCAB // 2, 2 * D))
    tlin = t2.reshape(VOCAB, D)
    out = _emb_lookup(xb, tlin)
    out5 = out.reshape(S, 8, NB, 8, 128)
    return out5.transpose(2, 4, 0, 1, 3).reshape(16384, S, D)


# speedup vs baseline: 1.0457x; 1.0457x over previous
import functools

import jax
import jax.numpy as jnp
from jax import lax
from jax.experimental import pallas as pl
from jax.experimental.pallas import tpu as pltpu
from jax.experimental.pallas import tpu_sc as plsc

VOCAB = 1000000
D = 64
S = 50
NB = 16384 // 128
NBLK = S * NB
NW = 32
BLK_PER_W = NBLK // NW
SCALE = float(D) ** 0.5

_MESH = plsc.VectorSubcoreMesh(core_axis_name="c", subcore_axis_name="s")


@functools.partial(
    pl.kernel,
    out_type=jax.ShapeDtypeStruct((S, 8, 1024, 128), jnp.float32),
    mesh=_MESH,
    compiler_params=pltpu.CompilerParams(
        use_tc_tiling_on_sc=False, needs_layout_passes=False),
    scratch_types=[
        pltpu.VMEM((BLK_PER_W, 128), jnp.int32),
        pltpu.VMEM((128, D), jnp.float32),
        pltpu.VMEM((128, D), jnp.float32),
        pltpu.VMEM((8, 8, 128), jnp.float32),
        pltpu.VMEM((8, 8, 128), jnp.float32),
        pltpu.SemaphoreType.DMA,
        pltpu.SemaphoreType.DMA,
        pltpu.SemaphoreType.DMA,
        pltpu.SemaphoreType.DMA,
    ],
)
def _emb_lookup(xb_hbm, table_hbm, out_hbm, idx_v, g0, g1, ob0, ob1,
                semg0, semg1, sems0, sems1):
    wid = lax.axis_index("s") * 2 + lax.axis_index("c")
    base_blk = wid * BLK_PER_W

    pltpu.sync_copy(xb_hbm.at[pl.ds(base_blk, BLK_PER_W)], idx_v)

    dvs = [lax.iota(jnp.int32, 16) + 16 * k for k in range(4)]
    ivs = [dv >> 3 for dv in dvs]
    qvs = [dv & 7 for dv in dvs]

    def process(gbuf, obuf, sems, n):
        blk = base_blk + n
        s = blk // NB
        j = blk % NB
        dst = out_hbm.at[s, :, pl.ds(8 * j, 8), :]

        @pl.when(n >= 2)
        def _():
            pltpu.make_async_copy(obuf, dst, sems).wait()

        def row(p, carry):
            pv = jnp.full((16,), p, jnp.int32)
            for k in range(4):
                v = gbuf[p, pl.ds(16 * k, 16)] * SCALE
                plsc.store_scatter(obuf, [ivs[k], qvs[k], pv], v)
            return carry

        lax.fori_loop(0, 128, row, 0, unroll=4)

        pltpu.async_copy(obuf, dst, sems)

    pltpu.async_copy(table_hbm.at[idx_v.at[0]], g0, semg0)

    def pair(g, carry):
        n0 = 2 * g
        pltpu.async_copy(table_hbm.at[idx_v.at[n0 + 1]], g1, semg1)
        pltpu.make_async_copy(table_hbm.at[idx_v.at[0]], g0, semg0).wait()
        process(g0, ob0, sems0, n0)
        nxt = jnp.minimum(n0 + 2, BLK_PER_W - 1)
        pltpu.async_copy(table_hbm.at[idx_v.at[nxt]], g0, semg0)
        pltpu.make_async_copy(table_hbm.at[idx_v.at[0]], g1, semg1).wait()
        process(g1, ob1, sems1, n0 + 1)
        return carry

    lax.fori_loop(0, BLK_PER_W // 2, pair, 0)

    pltpu.make_async_copy(table_hbm.at[idx_v.at[0]], g0, semg0).wait()
    drain_dst = out_hbm.at[0, :, pl.ds(0, 8), :]
    pltpu.make_async_copy(ob0, drain_dst, sems0).wait()
    pltpu.make_async_copy(ob1, drain_dst, sems1).wait()


def kernel(x, table):
    xb = x.T.reshape(S, NB, 128).reshape(NBLK, 128).astype(jnp.int32)
    t2 = jax.lax.optimization_barrier(table.reshape(VOCAB // 2, 2 * D))
    tlin = t2.reshape(VOCAB, D)
    out = _emb_lookup(xb, tlin)
    out5 = out.reshape(S, 8, 128, 8, 128)
    return out5.transpose(2, 4, 0, 1, 3).reshape(16384, S, D)


# speedup vs baseline: 1.0782x; 1.0311x over previous
import functools

import jax
import jax.numpy as jnp
import numpy as np
from jax import lax
from jax.experimental import pallas as pl
from jax.experimental.pallas import tpu as pltpu
from jax.experimental.pallas import tpu_sc as plsc

VOCAB = 1000000
D = 64
S = 50
NB = 16384 // 128
NBLK = S * NB
NW = 32
BLK_PER_W = NBLK // NW
SCALE = float(D) ** 0.5

_MESH = plsc.VectorSubcoreMesh(core_axis_name="c", subcore_axis_name="s")


def _repack_kernel(t_ref, o_ref):
    y = t_ref[...].T
    o_ref[...] = jnp.concatenate([y[:512], y[512:]], axis=1)


def _repack_table(tt):
    grid = (VOCAB + 1023) // 1024
    return pl.pallas_call(
        _repack_kernel,
        grid=(grid,),
        in_specs=[pl.BlockSpec((64, 1024), lambda c: (0, c))],
        out_specs=pl.BlockSpec((512, 128), lambda c: (c, 0)),
        out_shape=jax.ShapeDtypeStruct((VOCAB // 2, 128), jnp.float32),
    )(tt)


def _remap(r):
    off = r % 1024
    return (r - off) + 2 * (off % 512) + off // 512


@functools.partial(
    pl.kernel,
    out_type=jax.ShapeDtypeStruct((NBLK * 128, D), jnp.float32),
    mesh=_MESH,
    compiler_params=pltpu.CompilerParams(
        use_tc_tiling_on_sc=False, needs_layout_passes=False),
    scratch_types=[
        pltpu.VMEM((BLK_PER_W, 128), jnp.int32),
        pltpu.VMEM((128, D), jnp.float32),
        pltpu.VMEM((128, D), jnp.float32),
        pltpu.SemaphoreType.DMA,
        pltpu.SemaphoreType.DMA,
    ],
)
def _gather(xb_hbm, table_hbm, out_hbm, idx_v, g0, g1, semg0, semg1):
    wid = lax.axis_index("s") * 2 + lax.axis_index("c")
    base_blk = wid * BLK_PER_W

    pltpu.sync_copy(xb_hbm.at[pl.ds(base_blk, BLK_PER_W)], idx_v)

    def store(gbuf, n):
        pltpu.sync_copy(gbuf, out_hbm.at[pl.ds((base_blk + n) * 128, 128)])

    pltpu.async_copy(table_hbm.at[idx_v.at[0]], g0, semg0)

    def pair(g, carry):
        n0 = 2 * g
        pltpu.async_copy(table_hbm.at[idx_v.at[n0 + 1]], g1, semg1)
        pltpu.make_async_copy(table_hbm.at[idx_v.at[0]], g0, semg0).wait()
        store(g0, n0)
        nxt = jnp.minimum(n0 + 2, BLK_PER_W - 1)
        pltpu.async_copy(table_hbm.at[idx_v.at[nxt]], g0, semg0)
        pltpu.make_async_copy(table_hbm.at[idx_v.at[0]], g1, semg1).wait()
        store(g1, n0 + 1)
        return carry

    lax.fori_loop(0, BLK_PER_W // 2, pair, 0)

    pltpu.make_async_copy(table_hbm.at[idx_v.at[0]], g0, semg0).wait()


def _finalize_kernel(g_ref, o_ref):
    for jloc in range(8):
        x = g_ref[64 * jloc:64 * (jloc + 1), :]
        y = jnp.concatenate([x[:, :D].T, x[:, D:].T], axis=1) * SCALE
        o_ref[0, :, 8 * jloc:8 * (jloc + 1), :] = y.reshape(8, 8, 128)


def _finalize(gathered):
    return pl.pallas_call(
        _finalize_kernel,
        grid=(S, NB // 8),
        in_specs=[pl.BlockSpec((512, 128), lambda s, c: (16 * s + c, 0))],
        out_specs=pl.BlockSpec((1, 8, 64, 128), lambda s, c: (s, 0, c, 0)),
        out_shape=jax.ShapeDtypeStruct((S, 8, 1024, 128), jnp.float32),
    )(gathered)


_SLOT_PERM = np.arange(128)
_SLOT_PERM = 64 * (_SLOT_PERM % 2) + _SLOT_PERM // 2


def kernel(x, table):
    xt = x.T.reshape(S, NB, 128).astype(jnp.int32)
    xb = _remap(jnp.take(xt, jnp.asarray(_SLOT_PERM), axis=-1))
    xb = xb.reshape(NBLK, 128)

    tt = table.T
    t2 = _repack_table(tt)
    tlin = t2.reshape(VOCAB, D)
    g = _gather(xb, tlin)
    g2 = g.reshape(NBLK // 2 * 128, 128)
    out = _finalize(g2)
    out5 = out.reshape(S, 8, 128, 8, 128)
    return out5.transpose(2, 4, 0, 1, 3).reshape(16384, S, D)


# speedup vs baseline: 1.0795x; 1.0012x over previous
import functools

import jax
import jax.numpy as jnp
import numpy as np
from jax import lax
from jax.experimental import pallas as pl
from jax.experimental.pallas import tpu as pltpu
from jax.experimental.pallas import tpu_sc as plsc

VOCAB = 1000000
NCHUNK = (VOCAB + 1023) // 1024
VOCAB_PAD = NCHUNK * 1024
D = 64
S = 50
NB = 16384 // 128
NBLK = S * NB
NW = 32
BLK_PER_W = NBLK // NW
SCALE = float(D) ** 0.5

_MESH = plsc.VectorSubcoreMesh(core_axis_name="c", subcore_axis_name="s")


def _repack_kernel(t_ref, o_ref):
    y = t_ref[...].T
    o_ref[...] = jnp.concatenate([y[:512], y[512:]], axis=1)


def _repack_table(tt):
    return pl.pallas_call(
        _repack_kernel,
        grid=(NCHUNK,),
        in_specs=[pl.BlockSpec((64, 1024), lambda c: (0, c))],
        out_specs=pl.BlockSpec((512, 128), lambda c: (c, 0)),
        out_shape=jax.ShapeDtypeStruct((VOCAB_PAD // 2, 128), jnp.float32),
    )(tt)


def _remap(r):
    off = r % 1024
    return (r - off) + 2 * (off % 512) + off // 512


@functools.partial(
    pl.kernel,
    out_type=jax.ShapeDtypeStruct((NBLK * 128, D), jnp.float32),
    mesh=_MESH,
    compiler_params=pltpu.CompilerParams(
        use_tc_tiling_on_sc=False, needs_layout_passes=False),
    scratch_types=[
        pltpu.VMEM((BLK_PER_W, 128), jnp.int32),
        pltpu.VMEM((128, D), jnp.float32),
        pltpu.VMEM((128, D), jnp.float32),
        pltpu.SemaphoreType.DMA,
        pltpu.SemaphoreType.DMA,
    ],
)
def _gather(xb_hbm, table_hbm, out_hbm, idx_v, g0, g1, semg0, semg1):
    wid = lax.axis_index("s") * 2 + lax.axis_index("c")
    base_blk = wid * BLK_PER_W

    pltpu.sync_copy(xb_hbm.at[pl.ds(base_blk, BLK_PER_W)], idx_v)

    def store(gbuf, n):
        pltpu.sync_copy(gbuf, out_hbm.at[pl.ds((base_blk + n) * 128, 128)])

    pltpu.async_copy(table_hbm.at[idx_v.at[0]], g0, semg0)

    def pair(g, carry):
        n0 = 2 * g
        pltpu.async_copy(table_hbm.at[idx_v.at[n0 + 1]], g1, semg1)
        pltpu.make_async_copy(table_hbm.at[idx_v.at[0]], g0, semg0).wait()
        store(g0, n0)
        nxt = jnp.minimum(n0 + 2, BLK_PER_W - 1)
        pltpu.async_copy(table_hbm.at[idx_v.at[nxt]], g0, semg0)
        pltpu.make_async_copy(table_hbm.at[idx_v.at[0]], g1, semg1).wait()
        store(g1, n0 + 1)
        return carry

    lax.fori_loop(0, BLK_PER_W // 2, pair, 0)

    pltpu.make_async_copy(table_hbm.at[idx_v.at[0]], g0, semg0).wait()


def _finalize_kernel(g_ref, o_ref):
    for jloc in range(8):
        x = g_ref[64 * jloc:64 * (jloc + 1), :]
        y = jnp.concatenate([x[:, :D].T, x[:, D:].T], axis=1) * SCALE
        o_ref[0, :, 8 * jloc:8 * (jloc + 1), :] = y.reshape(8, 8, 128)


def _finalize(gathered):
    return pl.pallas_call(
        _finalize_kernel,
        grid=(S, NB // 8),
        in_specs=[pl.BlockSpec((512, 128), lambda s, c: (16 * s + c, 0))],
        out_specs=pl.BlockSpec((1, 8, 64, 128), lambda s, c: (s, 0, c, 0)),
        out_shape=jax.ShapeDtypeStruct((S, 8, 1024, 128), jnp.float32),
    )(gathered)


_SLOT_PERM = np.arange(128)
_SLOT_PERM = 64 * (_SLOT_PERM % 2) + _SLOT_PERM // 2


def kernel(x, table):
    xt = x.T.reshape(S, NB, 128).astype(jnp.int32)
    xb = _remap(jnp.take(xt, jnp.asarray(_SLOT_PERM), axis=-1))
    xb = xb.reshape(NBLK, 128)

    tt = table.T
    t2 = _repack_table(tt)
    tlin = t2.reshape(VOCAB_PAD, D)
    g = _gather(xb, tlin)
    g2 = g.reshape(NBLK // 2 * 128, 128)
    out = _finalize(g2)
    out5 = out.reshape(S, 8, 128, 8, 128)
    return out5.transpose(2, 4, 0, 1, 3).reshape(16384, S, D)
